# baseline (speedup 1.0000x reference)
import functools
import math

import jax
import jax.numpy as jnp
from jax import lax
from jax.experimental import pallas as pl
from jax.experimental.pallas import tpu as pltpu
from jax.experimental.pallas import tpu_sc as plsc

VOCAB = 1000000
D_MODEL = 64
ROWS = 4096
COLS = 200
B = ROWS * COLS
NC = 2
NS = 16
NW = NC * NS
BPW = B // NW
CHUNK = 800
NCHUNK = BPW // CHUNK
SCALE = math.sqrt(D_MODEL)

_mesh = plsc.VectorSubcoreMesh(core_axis_name="c", subcore_axis_name="s")


@functools.partial(
    pl.kernel,
    mesh=_mesh,
    out_type=jax.ShapeDtypeStruct((B, D_MODEL), jnp.float32),
    scratch_types=[
        pltpu.VMEM((CHUNK,), jnp.int32),
        pltpu.VMEM((CHUNK, D_MODEL), jnp.float32),
        pltpu.SemaphoreType.DMA,
    ],
    compiler_params=pltpu.CompilerParams(use_tc_tiling_on_sc=False),
)
def _embed(x_hbm, lut_hbm, out_hbm, idx_v, rows_v, sem):
    wid = lax.axis_index("s") * NC + lax.axis_index("c")
    base = wid * BPW

    def chunk_body(g, carry):
        off = base + g * CHUNK
        pltpu.sync_copy(x_hbm.at[pl.ds(off, CHUNK)], idx_v)
        pltpu.async_copy(lut_hbm.at[idx_v], rows_v, sem).wait()

        def row_body(r, c):
            for j in range(D_MODEL // 16):
                sl = pl.ds(j * 16, 16)
                rows_v[r, sl] = rows_v[r, sl] * SCALE
            return c

        lax.fori_loop(0, CHUNK, row_body, 0)
        pltpu.sync_copy(rows_v, out_hbm.at[pl.ds(off, CHUNK)])
        return carry

    lax.fori_loop(0, NCHUNK, chunk_body, 0)


def kernel(x, lut):
    out = _embed(x.reshape(B).astype(jnp.int32), lut)
    return out.reshape(ROWS, COLS, D_MODEL)


# speedup vs baseline: 1.1155x; 1.1155x over previous
import functools
import math

import jax
import jax.numpy as jnp
from jax import lax
from jax.experimental import pallas as pl
from jax.experimental.pallas import tpu as pltpu
from jax.experimental.pallas import tpu_sc as plsc

VOCAB = 1000000
D_MODEL = 64
ROWS = 4096
COLS = 200
B = ROWS * COLS
NC = 2
NS = 16
NW = NC * NS
BPW = B // NW
CHUNK = 400
NCH = BPW // CHUNK
NBUF = 4
SCALE = math.sqrt(D_MODEL)

_mesh = plsc.VectorSubcoreMesh(core_axis_name="c", subcore_axis_name="s")


@functools.partial(
    pl.kernel,
    mesh=_mesh,
    out_type=jax.ShapeDtypeStruct((B, D_MODEL), jnp.float32),
    scratch_types=[pltpu.VMEM((BPW,), jnp.int32)]
    + [pltpu.VMEM((CHUNK, D_MODEL), jnp.float32)] * NBUF
    + [pltpu.SemaphoreType.DMA] * (2 * NBUF),
    compiler_params=pltpu.CompilerParams(use_tc_tiling_on_sc=False),
)
def _embed(x_hbm, lut_hbm, out_hbm, idx_v,
           r0, r1, r2, r3, g0, g1, g2, g3, s0, s1, s2, s3):
    rows = (r0, r1, r2, r3)
    gsem = (g0, g1, g2, g3)
    ssem = (s0, s1, s2, s3)
    wid = lax.axis_index("s") * NC + lax.axis_index("c")
    base = wid * BPW
    pltpu.sync_copy(x_hbm.at[pl.ds(base, BPW)], idx_v)

    def start_gather(g, b):
        pltpu.async_copy(
            lut_hbm.at[idx_v.at[pl.ds(g * CHUNK, CHUNK)]], rows[b], gsem[b])

    def wait_gather(b):
        pltpu.make_async_copy(
            lut_hbm.at[idx_v.at[pl.ds(0, CHUNK)]], rows[b], gsem[b]).wait()

    def wait_store(b):
        pltpu.make_async_copy(
            rows[b], out_hbm.at[pl.ds(0, CHUNK)], ssem[b]).wait()

    def scale(buf):
        def body(i, c):
            r = i * 4
            for k in range(4):
                for j in range(D_MODEL // 16):
                    sl = pl.ds(j * 16, 16)
                    buf[r + k, sl] = buf[r + k, sl] * SCALE
            return c
        lax.fori_loop(0, CHUNK // 4, body, 0)

    for b in range(NBUF - 1):
        start_gather(b, b)

    def group(t, carry):
        for bb in range(NBUF):
            g = t * NBUF + bb
            gl = g + NBUF - 1
            bl = (bb + NBUF - 1) % NBUF

            @pl.when(gl < NCH)
            def _():
                @pl.when(gl >= NBUF)
                def _():
                    wait_store(bl)
                start_gather(gl, bl)

            wait_gather(bb)
            scale(rows[bb])
            pltpu.async_copy(
                rows[bb], out_hbm.at[pl.ds(base + g * CHUNK, CHUNK)], ssem[bb])
        return carry

    lax.fori_loop(0, NCH // NBUF, group, 0)
    for b in range(NBUF):
        wait_store(b)


def kernel(x, lut):
    out = _embed(x.reshape(B).astype(jnp.int32), lut)
    return out.reshape(ROWS, COLS, D_MODEL)


# speedup vs baseline: 1.1186x; 1.0028x over previous
import functools
import math

import jax
import jax.numpy as jnp
from jax import lax
from jax.experimental import pallas as pl
from jax.experimental.pallas import tpu as pltpu
from jax.experimental.pallas import tpu_sc as plsc

VOCAB = 1000000
D_MODEL = 64
ROWS = 4096
COLS = 200
B = ROWS * COLS
NC = 2
NS = 16
NW = NC * NS
XPW = ROWS // NW
BPW = B // NW
CHUNK = COLS
NCH = XPW
NBUF = 4
SCALE = math.sqrt(D_MODEL)

_mesh = plsc.VectorSubcoreMesh(core_axis_name="c", subcore_axis_name="s")


@functools.partial(
    pl.kernel,
    mesh=_mesh,
    out_type=jax.ShapeDtypeStruct((ROWS, COLS, D_MODEL), jnp.float32),
    scratch_types=[pltpu.VMEM((BPW,), jnp.int32)]
    + [pltpu.VMEM((CHUNK, D_MODEL), jnp.float32)] * NBUF
    + [pltpu.SemaphoreType.DMA] * (2 * NBUF),
    compiler_params=pltpu.CompilerParams(use_tc_tiling_on_sc=False),
)
def _embed(x_hbm, lut_hbm, out_hbm, idx_v,
           r0, r1, r2, r3, g0, g1, g2, g3, s0, s1, s2, s3):
    rows = (r0, r1, r2, r3)
    gsem = (g0, g1, g2, g3)
    ssem = (s0, s1, s2, s3)
    wid = lax.axis_index("s") * NC + lax.axis_index("c")
    xbase = wid * XPW
    pltpu.sync_copy(x_hbm.at[pl.ds(wid * BPW, BPW)], idx_v)

    def start_gather(g, b):
        pltpu.async_copy(
            lut_hbm.at[idx_v.at[pl.ds(g * CHUNK, CHUNK)]], rows[b], gsem[b])

    def wait_gather(b):
        pltpu.make_async_copy(
            lut_hbm.at[idx_v.at[pl.ds(0, CHUNK)]], rows[b], gsem[b]).wait()

    def wait_store(b):
        pltpu.make_async_copy(rows[b], out_hbm.at[xbase], ssem[b]).wait()

    def scale(buf):
        def body(i, c):
            r = i * 4
            for k in range(4):
                for j in range(D_MODEL // 16):
                    sl = pl.ds(j * 16, 16)
                    buf[r + k, sl] = buf[r + k, sl] * SCALE
            return c
        lax.fori_loop(0, CHUNK // 4, body, 0)

    for b in range(NBUF - 1):
        start_gather(b, b)

    def group(t, carry):
        for bb in range(NBUF):
            g = t * NBUF + bb
            gl = g + NBUF - 1
            bl = (bb + NBUF - 1) % NBUF

            @pl.when(gl < NCH)
            def _():
                @pl.when(gl >= NBUF)
                def _():
                    wait_store(bl)
                start_gather(gl, bl)

            wait_gather(bb)
            scale(rows[bb])
            pltpu.async_copy(rows[bb], out_hbm.at[xbase + g], ssem[bb])
        return carry

    lax.fori_loop(0, NCH // NBUF, group, 0)
    for b in range(NBUF):
        wait_store(b)


def kernel(x, lut):
    return _embed(x.reshape(B), lut)
